# baseline (speedup 1.0000x reference)
import functools

import jax
import jax.numpy as jnp
from jax import lax
from jax.experimental import pallas as pl
from jax.experimental.pallas import tpu as pltpu
from jax.experimental.pallas import tpu_sc as plsc

_VOCAB = 1000000
_EMB = 64
_PAD = 128
_BATCH = 16384
_HIST = 200
_NW = 32
_LB = 128
_NBC = _BATCH // _LB
_BCW = _NBC // _NW
_NBLK = _BCW * _HIST
_GBUF = 4
_NBUF = 2


def _make_lookup():
    mesh = plsc.VectorSubcoreMesh(core_axis_name="c", subcore_axis_name="s")

    @functools.partial(
        pl.kernel,
        mesh=mesh,
        out_type=jax.ShapeDtypeStruct((_HIST, _EMB, _BATCH), jnp.float32),
        scratch_types=[
            pltpu.VMEM((_LB,), jnp.int32),
            pltpu.VMEM((_LB,), jnp.int32),
            pltpu.VMEM((_LB,), jnp.int32),
            pltpu.VMEM((_LB,), jnp.int32),
            pltpu.VMEM((_LB, _PAD), jnp.float32),
            pltpu.VMEM((_LB, _PAD), jnp.float32),
            pltpu.VMEM((_LB, _PAD), jnp.float32),
            pltpu.VMEM((_LB, _PAD), jnp.float32),
            pltpu.VMEM((_EMB, _LB), jnp.float32),
            pltpu.VMEM((_EMB, _LB), jnp.float32),
            pltpu.SemaphoreType.DMA,
            pltpu.SemaphoreType.DMA,
            pltpu.SemaphoreType.DMA,
            pltpu.SemaphoreType.DMA,
            pltpu.SemaphoreType.DMA,
            pltpu.SemaphoreType.DMA,
            pltpu.SemaphoreType.DMA,
            pltpu.SemaphoreType.DMA,
            pltpu.SemaphoreType.DMA,
            pltpu.SemaphoreType.DMA,
        ],
        compiler_params=pltpu.CompilerParams(use_tc_tiling_on_sc=True,
                                             needs_layout_passes=False,
                                             disable_bounds_checks=True),
    )
    def lookup(idx_hbm, table_hbm, out_hbm,
               i0, i1, i2, i3, r0, r1, r2, r3, t0, t1,
               si0, si1, si2, si3, sg0, sg1, sg2, sg3, ss0, ss1):
        idx_v = (i0, i1, i2, i3)
        rows_v = (r0, r1, r2, r3)
        tout_v = (t0, t1)
        sem_i = (si0, si1, si2, si3)
        sem_g = (sg0, sg1, sg2, sg3)
        sem_s = (ss0, ss1)
        wid = lax.axis_index("s") * 2 + lax.axis_index("c")
        bc0 = wid * _BCW

        dvecs = [jnp.arange(16, dtype=jnp.int32) + 16 * dg for dg in range(4)]
        lanes = jnp.arange(16, dtype=jnp.int32)

        def hb(j):
            return j % _HIST, bc0 + j // _HIST

        def idx_src(j):
            h, bc = hb(j)
            return idx_hbm.at[h, pl.ds(bc * _LB, _LB)]

        def out_dst(j):
            h, bc = hb(j)
            return out_hbm.at[h, pl.ds(0, _EMB), pl.ds(bc * _LB, _LB)]

        def transpose_block(b, tb):
            for bl in range(0, _LB, 2):
                pairs = [(bl + i, dg) for i in range(2) for dg in range(4)]
                bvecs = {p: (lanes + p) & (_LB - 1) for p in (bl, bl + 1)}
                vals = [plsc.load_gather(rows_v[b], [bvecs[p], dvecs[dg]])
                        for p, dg in pairs]
                for (p, dg), v in zip(pairs, vals):
                    plsc.store_scatter(tout_v[tb], [dvecs[dg], bvecs[p]], v)

        for s in range(_GBUF):
            pltpu.async_copy(idx_src(s), idx_v[s], sem_i[s])
        for s in range(_GBUF - 1):
            pltpu.make_async_copy(idx_src(s), idx_v[s], sem_i[s]).wait()
            pltpu.async_copy(table_hbm.at[idx_v[s]], rows_v[s], sem_g[s])

        def body(j0, carry):
            for k in range(_GBUF):
                j = j0 * _GBUF + k
                tb = k % _NBUF
                pltpu.make_async_copy(table_hbm.at[idx_v[k]],
                                      rows_v[k], sem_g[k]).wait()
                @pl.when(j + _GBUF < _NBLK)
                def _():
                    pltpu.async_copy(idx_src(j + _GBUF), idx_v[k], sem_i[k])
                kg = (k + _GBUF - 1) % _GBUF
                @pl.when(j + _GBUF - 1 < _NBLK)
                def _():
                    pltpu.make_async_copy(idx_src(j + _GBUF - 1),
                                          idx_v[kg], sem_i[kg]).wait()
                    pltpu.async_copy(table_hbm.at[idx_v[kg]],
                                     rows_v[kg], sem_g[kg])
                @pl.when(j >= _NBUF)
                def _():
                    pltpu.make_async_copy(tout_v[tb], out_dst(0),
                                          sem_s[tb]).wait()
                transpose_block(k, tb)
                pltpu.async_copy(tout_v[tb], out_dst(j), sem_s[tb])
            return carry

        lax.fori_loop(0, _NBLK // _GBUF, body, 0)

        for tb in range(_NBUF):
            pltpu.make_async_copy(tout_v[tb], out_dst(0), sem_s[tb]).wait()

    return lookup


_lookup = _make_lookup()


@jax.jit
def kernel(indices, table):
    table_p = jnp.pad(table, ((0, 0), (0, _PAD - _EMB)))
    out_t = _lookup(indices.T, table_p)
    return jnp.transpose(out_t, (2, 0, 1))


# speedup vs baseline: 1.5073x; 1.5073x over previous
import functools

import jax
import jax.numpy as jnp
from jax import lax
from jax.experimental import pallas as pl
from jax.experimental.pallas import tpu as pltpu
from jax.experimental.pallas import tpu_sc as plsc

_VOCAB = 1000000
_EMB = 64
_PAD = 128
_BATCH = 16384
_HIST = 200
_B = _BATCH * _HIST
_NW = 32
_BPW = _B // _NW
_CHUNK = 400
_NCHUNK = _BPW // _CHUNK
_NBUF = 2
_QB = _CHUNK // _HIST


def _make_lookup():
    mesh = plsc.VectorSubcoreMesh(core_axis_name="c", subcore_axis_name="s")

    @functools.partial(
        pl.kernel,
        mesh=mesh,
        out_type=jax.ShapeDtypeStruct((_BATCH, _HIST, _PAD), jnp.float32),
        scratch_types=[
            pltpu.VMEM((_CHUNK,), jnp.int32),
            pltpu.VMEM((_CHUNK,), jnp.int32),
            pltpu.VMEM((_CHUNK, _PAD), jnp.float32),
            pltpu.VMEM((_CHUNK, _PAD), jnp.float32),
            pltpu.SemaphoreType.DMA,
            pltpu.SemaphoreType.DMA,
            pltpu.SemaphoreType.DMA,
            pltpu.SemaphoreType.DMA,
            pltpu.SemaphoreType.DMA,
            pltpu.SemaphoreType.DMA,
        ],
        compiler_params=pltpu.CompilerParams(use_tc_tiling_on_sc=True),
    )
    def lookup(idx_hbm, table_hbm, out_hbm, idx_v0, idx_v1, rows_v0, rows_v1,
               sem_i0, sem_i1, sem_g0, sem_g1, sem_s0, sem_s1):
        idx_v = (idx_v0, idx_v1)
        rows_v = (rows_v0, rows_v1)
        sem_i = (sem_i0, sem_i1)
        sem_g = (sem_g0, sem_g1)
        sem_s = (sem_s0, sem_s1)
        wid = lax.axis_index("s") * 2 + lax.axis_index("c")
        base = wid * _BPW

        for b in range(_NBUF):
            pltpu.async_copy(
                idx_hbm.at[pl.ds(base + b * _CHUNK, _CHUNK)],
                idx_v[b], sem_i[b])

        def body(i, carry):
            g0 = i * _NBUF
            for b in range(_NBUF):
                g = g0 + b
                off = base + g * _CHUNK
                pltpu.make_async_copy(
                    idx_hbm.at[pl.ds(off, _CHUNK)], idx_v[b],
                    sem_i[b]).wait()
                @pl.when(g0 > 0)
                def _():
                    for q in range(_QB):
                        pltpu.make_async_copy(
                            rows_v[b].at[pl.ds(q * _HIST, _HIST)],
                            out_hbm.at[0], sem_s[b]).wait()
                pltpu.async_copy(table_hbm.at[idx_v[b]], rows_v[b],
                                 sem_g[b])
                pltpu.make_async_copy(table_hbm.at[idx_v[b]],
                                      rows_v[b], sem_g[b]).wait()
                @pl.when(g + _NBUF < _NCHUNK)
                def _():
                    pltpu.async_copy(
                        idx_hbm.at[pl.ds(off + _NBUF * _CHUNK, _CHUNK)],
                        idx_v[b], sem_i[b])
                bat0 = off // _HIST
                for q in range(_QB):
                    pltpu.async_copy(
                        rows_v[b].at[pl.ds(q * _HIST, _HIST)],
                        out_hbm.at[bat0 + q], sem_s[b])
            return carry

        lax.fori_loop(0, _NCHUNK // _NBUF, body, 0)

        for b in range(_NBUF):
            for q in range(_QB):
                pltpu.make_async_copy(
                    rows_v[b].at[pl.ds(q * _HIST, _HIST)],
                    out_hbm.at[0], sem_s[b]).wait()

    return lookup


_lookup = _make_lookup()


@jax.jit
def kernel(indices, table):
    table_p = jnp.pad(table, ((0, 0), (0, _PAD - _EMB)))
    out = _lookup(indices.reshape(_B), table_p)
    return out[:, :, :_EMB]


# speedup vs baseline: 1.5107x; 1.0022x over previous
import functools

import jax
import jax.numpy as jnp
from jax import lax
from jax.experimental import pallas as pl
from jax.experimental.pallas import tpu as pltpu
from jax.experimental.pallas import tpu_sc as plsc

_VOCAB = 1000000
_EMB = 64
_PAD = 128
_BATCH = 16384
_HIST = 200
_B = _BATCH * _HIST
_NW = 32
_BPW = _B // _NW
_CHUNK = 200
_NCHUNK = _BPW // _CHUNK
_GBUF = 4


def _make_lookup():
    mesh = plsc.VectorSubcoreMesh(core_axis_name="c", subcore_axis_name="s")

    @functools.partial(
        pl.kernel,
        mesh=mesh,
        out_type=jax.ShapeDtypeStruct((_BATCH, _HIST, _PAD), jnp.float32),
        scratch_types=(
            [pltpu.VMEM((_CHUNK,), jnp.int32) for _ in range(_GBUF)]
            + [pltpu.VMEM((_CHUNK, _PAD), jnp.float32) for _ in range(_GBUF)]
            + [pltpu.SemaphoreType.DMA for _ in range(3 * _GBUF)]
        ),
        compiler_params=pltpu.CompilerParams(use_tc_tiling_on_sc=True),
    )
    def lookup(idx_hbm, table_hbm, out_hbm, *bufs):
        idx_v = bufs[:_GBUF]
        rows_v = bufs[_GBUF:2 * _GBUF]
        sem_i = bufs[2 * _GBUF:3 * _GBUF]
        sem_g = bufs[3 * _GBUF:4 * _GBUF]
        sem_s = bufs[4 * _GBUF:5 * _GBUF]
        wid = lax.axis_index("s") * 2 + lax.axis_index("c")
        base = wid * _BPW

        def idx_src(g):
            return idx_hbm.at[pl.ds(base + g * _CHUNK, _CHUNK)]

        def out_dst(g):
            return out_hbm.at[(base + g * _CHUNK) // _HIST]

        for s in range(_GBUF):
            pltpu.async_copy(idx_src(s), idx_v[s], sem_i[s])
        for s in range(2):
            pltpu.make_async_copy(idx_src(s), idx_v[s], sem_i[s]).wait()
            pltpu.async_copy(table_hbm.at[idx_v[s]], rows_v[s], sem_g[s])

        def body(g0, carry):
            for k in range(_GBUF):
                g = g0 * _GBUF + k
                pltpu.make_async_copy(table_hbm.at[idx_v[k]],
                                      rows_v[k], sem_g[k]).wait()
                pltpu.async_copy(rows_v[k], out_dst(g), sem_s[k])
                @pl.when(g + _GBUF < _NCHUNK)
                def _():
                    pltpu.async_copy(idx_src(g + _GBUF), idx_v[k], sem_i[k])
                kg = (k + 2) % _GBUF
                @pl.when(g + 2 < _NCHUNK)
                def _():
                    @pl.when(g >= 2)
                    def _():
                        pltpu.make_async_copy(rows_v[kg], out_dst(0),
                                              sem_s[kg]).wait()
                    pltpu.make_async_copy(idx_src(g + 2), idx_v[kg],
                                          sem_i[kg]).wait()
                    pltpu.async_copy(table_hbm.at[idx_v[kg]],
                                     rows_v[kg], sem_g[kg])
            return carry

        lax.fori_loop(0, _NCHUNK // _GBUF, body, 0)

        for k in range(_GBUF):
            pltpu.make_async_copy(rows_v[k], out_dst(0), sem_s[k]).wait()

    return lookup


_lookup = _make_lookup()


@jax.jit
def kernel(indices, table):
    table_p = jnp.pad(table, ((0, 0), (0, _PAD - _EMB)))
    out = _lookup(indices.reshape(_B), table_p)
    return out[:, :, :_EMB]
